# baseline (speedup 1.0000x reference)
import functools

import jax
import jax.numpy as jnp
import numpy as np
from jax import lax
from jax.experimental import pallas as pl
from jax.experimental.pallas import tpu as pltpu
from jax.experimental.pallas import tpu_sc as plsc

N = 10000
DIM = 128
NHEADS = 8
DH = DIM // NHEADS
NUM_LAYER = 2
E = 320000
R = 1000
B_ALIGN = 4096
B_REL = 512
T = 100000
ALPHA = 0.2

NC = 2
NS = 16
NW = NC * NS
K = 128

NP = 10112
ACC_W = 144
ETOT = E + N
KE = 64
EB = 162
EPW = EB * KE
EPAD = EPW * NW
TB = 26
TPW = TB * K
TP = TPW * NW

_INV3SQ = float(1.0 / (3.0 * np.sqrt(float(DIM))))


def _lane_bcast(v, idx16):
    return lax.gather(
        v, idx16[:, None],
        lax.GatherDimensionNumbers(offset_dims=(), collapsed_slice_dims=(0,),
                                   start_index_map=(0,)),
        (1,), mode=lax.GatherScatterMode.PROMISE_IN_BOUNDS)


def _proj_body(x_ref, wcat_ref, u1_ref, u2_ref, wh_ref, s1_ref, s2_ref):
    x = x_ref[...]
    wh_ref[...] = jnp.dot(x, wcat_ref[...], preferred_element_type=jnp.float32)
    s1_ref[...] = jnp.dot(x, u1_ref[...], preferred_element_type=jnp.float32)
    s2_ref[...] = jnp.dot(x, u2_ref[...], preferred_element_type=jnp.float32)


def _proj(x, wcat, u1, u2):
    grid = 4
    b = NP // grid
    return pl.pallas_call(
        _proj_body,
        grid=(grid,),
        in_specs=[
            pl.BlockSpec((b, DIM), lambda i: (i, 0)),
            pl.BlockSpec((DIM, DIM), lambda i: (0, 0)),
            pl.BlockSpec((DIM, 16), lambda i: (0, 0)),
            pl.BlockSpec((DIM, 16), lambda i: (0, 0)),
        ],
        out_specs=[
            pl.BlockSpec((b, DIM), lambda i: (i, 0)),
            pl.BlockSpec((b, 16), lambda i: (i, 0)),
            pl.BlockSpec((b, 16), lambda i: (i, 0)),
        ],
        out_shape=[
            jax.ShapeDtypeStruct((NP, DIM), jnp.float32),
            jax.ShapeDtypeStruct((NP, 16), jnp.float32),
            jax.ShapeDtypeStruct((NP, 16), jnp.float32),
        ],
    )(x, wcat, u1, u2)


def _rep8():
    row = lax.broadcasted_iota(jnp.int32, (NHEADS, DIM), 0)
    lane = lax.broadcasted_iota(jnp.int32, (NHEADS, DIM), 1) % NHEADS
    return (row == lane).astype(jnp.float32)


def _combine_elu_proj_body(acc_ref, wcat_ref, u1_ref, u2_ref,
                           wh_ref, s1_ref, s2_ref):
    a = acc_ref[0] + acc_ref[1]
    drep = jnp.dot(a[:, DIM:DIM + NHEADS], _rep8(),
                   preferred_element_type=jnp.float32)
    x = a[:, :DIM] / (drep + 1e-16)
    x = jnp.where(x > 0, x, jnp.exp(x) - 1.0)
    wh_ref[...] = jnp.dot(x, wcat_ref[...], preferred_element_type=jnp.float32)
    s1_ref[...] = jnp.dot(x, u1_ref[...], preferred_element_type=jnp.float32)
    s2_ref[...] = jnp.dot(x, u2_ref[...], preferred_element_type=jnp.float32)


def _combine_elu_proj(acc, wcat, u1, u2):
    grid = 4
    b = NP // grid
    return pl.pallas_call(
        _combine_elu_proj_body,
        grid=(grid,),
        in_specs=[
            pl.BlockSpec((2, b, ACC_W), lambda i: (0, i, 0)),
            pl.BlockSpec((DIM, DIM), lambda i: (0, 0)),
            pl.BlockSpec((DIM, 16), lambda i: (0, 0)),
            pl.BlockSpec((DIM, 16), lambda i: (0, 0)),
        ],
        out_specs=[
            pl.BlockSpec((b, DIM), lambda i: (i, 0)),
            pl.BlockSpec((b, 16), lambda i: (i, 0)),
            pl.BlockSpec((b, 16), lambda i: (i, 0)),
        ],
        out_shape=[
            jax.ShapeDtypeStruct((NP, DIM), jnp.float32),
            jax.ShapeDtypeStruct((NP, 16), jnp.float32),
            jax.ShapeDtypeStruct((NP, 16), jnp.float32),
        ],
    )(acc, wcat, u1, u2)


def _combine_norm_body(acc_ref, g_ref):
    a = acc_ref[0] + acc_ref[1]
    drep = jnp.dot(a[:, DIM:DIM + NHEADS], _rep8(),
                   preferred_element_type=jnp.float32)
    g = a[:, :DIM] / (drep + 1e-16)
    rowj = lax.broadcasted_iota(jnp.int32, (DIM, DIM), 0)
    colo = lax.broadcasted_iota(jnp.int32, (DIM, DIM), 1)
    pm = (colo == (rowj % NHEADS) * DH + rowj // NHEADS).astype(jnp.float32)
    g = jnp.dot(g, pm, preferred_element_type=jnp.float32)
    nrm = jnp.sqrt(jnp.sum(g * g, axis=1, keepdims=True))
    g_ref[...] = g / (nrm + 1e-12)


def _combine_norm(acc):
    grid = 4
    b = NP // grid
    return pl.pallas_call(
        _combine_norm_body,
        grid=(grid,),
        in_specs=[pl.BlockSpec((2, b, ACC_W), lambda i: (0, i, 0))],
        out_specs=pl.BlockSpec((b, DIM), lambda i: (i, 0)),
        out_shape=jax.ShapeDtypeStruct((NP, DIM), jnp.float32),
    )(acc)


def _relnorm_body(x_ref, o_ref):
    x = x_ref[...]
    nrm = jnp.sqrt(jnp.sum(x * x, axis=1, keepdims=True))
    o_ref[...] = x / (nrm + 1e-12)


def _relnorm(x):
    return pl.pallas_call(
        _relnorm_body,
        out_shape=jax.ShapeDtypeStruct(x.shape, jnp.float32),
    )(x)


def _edge_pass(pe, wh, s1, s2):
    mesh = plsc.VectorSubcoreMesh(core_axis_name="c", subcore_axis_name="s")

    @functools.partial(
        pl.kernel,
        out_type=jax.ShapeDtypeStruct((NC, NP, ACC_W), jnp.float32),
        mesh=mesh,
        scratch_types=[
            [pltpu.VMEM((2 * KE,), jnp.int32)] * 2,
            [pltpu.VMEM((KE,), jnp.int32)] * 2,
            [pltpu.VMEM((KE, 16), jnp.float32)] * 2,
            [pltpu.VMEM((KE, 16), jnp.float32)] * 2,
            [pltpu.VMEM((KE, DIM), jnp.float32)] * 2,
            [pltpu.VMEM((KE, ACC_W), jnp.float32)] * 2,
            [pltpu.SemaphoreType.DMA] * 2,
            [pltpu.SemaphoreType.DMA] * 2,
            [pltpu.SemaphoreType.DMA] * 2,
            pltpu.VMEM_SHARED((NP, ACC_W), jnp.float32),
        ],
        compiler_params=pltpu.CompilerParams(use_tc_tiling_on_sc=False,
                                             needs_layout_passes=False),
    )
    def body(pe_ref, wh_ref, s1_ref, s2_ref, out_ref,
             ibuf, sidx, ssb, sdb, gwb, obb, isem, gsem, ssem, acc):
        c = lax.axis_index("c")
        s = lax.axis_index("s")
        wid = c * NS + s

        def zrow(e, carry):
            for j in range(ACC_W // 16):
                obb[0][e, pl.ds(j * 16, 16)] = jnp.zeros((16,), jnp.float32)
            return carry

        lax.fori_loop(0, KE, zrow, 0)

        row0 = s * (NP // NS)
        for off in range(0, NP // NS, KE):
            nr = min(KE, NP // NS - off)
            pltpu.sync_copy(obb[0].at[pl.ds(0, nr)],
                            acc.at[pl.ds(row0 + off, nr)])
        plsc.subcore_barrier()

        lanes = lax.broadcasted_iota(jnp.int32, (16,), 0)
        wstart = wid * EB * 2 * KE

        def idx_fetch(b, p):
            pltpu.async_copy(pe_ref.at[pl.ds(wstart + b * 2 * KE, 2 * KE)],
                             ibuf[p], isem[p])

        def idx_wait(p):
            pltpu.make_async_copy(pe_ref.at[pl.ds(0, 2 * KE)], ibuf[p],
                                  isem[p]).wait()

        def fire_gathers(p):
            pltpu.async_copy(s1_ref.at[ibuf[p].at[pl.ds(0, KE)]],
                             ssb[p], gsem[p])
            pltpu.async_copy(s2_ref.at[ibuf[p].at[pl.ds(KE, KE)]],
                             sdb[p], gsem[p])
            pltpu.async_copy(wh_ref.at[ibuf[p].at[pl.ds(0, KE)]],
                             gwb[p], gsem[p])

        def wait_gathers(p):
            pltpu.make_async_copy(s1_ref.at[ibuf[p].at[pl.ds(0, KE)]],
                                  ssb[p], gsem[p]).wait()
            pltpu.make_async_copy(s2_ref.at[ibuf[p].at[pl.ds(KE, KE)]],
                                  sdb[p], gsem[p]).wait()
            pltpu.make_async_copy(wh_ref.at[ibuf[p].at[pl.ds(0, KE)]],
                                  gwb[p], gsem[p]).wait()

        pltpu.sync_copy(pe_ref.at[pl.ds(wstart, 2 * KE)], ibuf[0])
        fire_gathers(0)
        idx_fetch(1, 1)

        def blk2(i, carry):
            for p in range(2):
                q = 1 - p
                b = 2 * i + p
                wait_gathers(p)

                @pl.when(b >= 2)
                def _():
                    pltpu.make_async_copy(
                        obb[p], acc.at[sidx[p]], ssem[p]).wait()

                for e in range(KE):
                    t = ssb[p][e, :] + sdb[p][e, :]
                    w = jnp.exp(jnp.where(t > 0, t, ALPHA * t))
                    obb[p][e, pl.ds(DIM, 16)] = jnp.where(
                        lanes < NHEADS, w, 0.0)
                    for ch in range(DIM // 16):
                        obb[p][e, pl.ds(ch * 16, 16)] = (
                            gwb[p][e, pl.ds(ch * 16, 16)] * w)

                for j in range(KE // 16):
                    sidx[p][pl.ds(j * 16, 16)] = ibuf[p][pl.ds(KE + j * 16,
                                                               16)]
                pltpu.async_copy(obb[p], acc.at[sidx[p]], ssem[p], add=True)

                @pl.when(b + 2 < EB)
                def _():
                    idx_fetch(b + 2, p)

                @pl.when(b + 1 < EB)
                def _():
                    idx_wait(q)
                    fire_gathers(q)
            return carry

        lax.fori_loop(0, EB // 2, blk2, 0)
        for p in range(2):
            pltpu.make_async_copy(obb[p], acc.at[sidx[p]], ssem[p]).wait()
        plsc.subcore_barrier()
        nr = NP // NS
        pltpu.sync_copy(acc.at[pl.ds(row0, nr)], out_ref.at[c, pl.ds(row0, nr)])

    return body(pe, wh, s1, s2)


def _transe_gather(g, rnorm, ti, d_idx, rl_idx):
    mesh = plsc.VectorSubcoreMesh(core_axis_name="c", subcore_axis_name="s")

    @functools.partial(
        pl.kernel,
        out_type=(
            jax.ShapeDtypeStruct((TP,), jnp.float32),
            jax.ShapeDtypeStruct((B_ALIGN, DIM), jnp.float32),
            jax.ShapeDtypeStruct((B_REL, DIM), jnp.float32),
        ),
        mesh=mesh,
        scratch_types=[
            pltpu.VMEM((TB * 3 * K,), jnp.int32),
            [pltpu.VMEM((K, DIM), jnp.float32)] * 2,
            [pltpu.VMEM((K, DIM), jnp.float32)] * 2,
            [pltpu.VMEM((K, DIM), jnp.float32)] * 2,
            [pltpu.SemaphoreType.DMA] * 2,
            pltpu.VMEM((K,), jnp.float32),
            pltpu.VMEM((16,), jnp.int32),
            pltpu.VMEM((16, DIM), jnp.float32),
        ],
        compiler_params=pltpu.CompilerParams(use_tc_tiling_on_sc=False,
                                             needs_layout_passes=False),
    )
    def body(g_ref, r_ref, ti_ref, di_ref, ri_ref,
             tv_ref, dout_ref, rout_ref,
             tib, gh, gt, gr, gsem, tvb, rlx, rbb):
        c = lax.axis_index("c")
        s = lax.axis_index("s")
        wid = c * NS + s

        pltpu.sync_copy(ti_ref.at[pl.ds(wid * TB * 3 * K, TB * 3 * K)], tib)

        def fire(b, p):
            off = pl.multiple_of(b * 3 * K, 128)
            pltpu.async_copy(g_ref.at[tib.at[pl.ds(off, K)]], gh[p], gsem[p])
            pltpu.async_copy(g_ref.at[tib.at[pl.ds(off + K, K)]],
                             gt[p], gsem[p])
            pltpu.async_copy(r_ref.at[tib.at[pl.ds(off + 2 * K, K)]],
                             gr[p], gsem[p])

        def wait_g(p):
            pltpu.make_async_copy(g_ref.at[tib.at[pl.ds(0, K)]],
                                  gh[p], gsem[p]).wait()
            pltpu.make_async_copy(g_ref.at[tib.at[pl.ds(0, K)]],
                                  gt[p], gsem[p]).wait()
            pltpu.make_async_copy(r_ref.at[tib.at[pl.ds(0, K)]],
                                  gr[p], gsem[p]).wait()

        fire(0, 0)
        lanes = lax.broadcasted_iota(jnp.int32, (16,), 0)

        def blk2(i, carry):
            for p in range(2):
                q = 1 - p
                b = 2 * i + p
                wait_g(p)

                @pl.when(b + 1 < TB)
                def _():
                    fire(b + 1, q)

                lane15 = jnp.full((16,), 15, jnp.int32)

                def tri16(g16, ecarry):
                    res = jnp.zeros((16,), jnp.float32)
                    for j in range(16):
                        e = g16 * 16 + j
                        acc = jnp.zeros((16,), jnp.float32)
                        for ch in range(DIM // 16):
                            sl = pl.ds(ch * 16, 16)
                            acc = acc + jnp.abs(
                                gh[p][e, sl] + gr[p][e, sl] - gt[p][e, sl])
                        bsum = _lane_bcast(plsc.cumsum(acc), lane15)
                        res = jnp.where(lanes == j,
                                        1.0 - bsum * _INV3SQ, res)
                    tvb[pl.ds(g16 * 16, 16)] = res
                    return ecarry

                pass
                pltpu.sync_copy(tvb, tv_ref.at[pl.ds(wid * TPW + b * K, K)])
            return carry

        lax.fori_loop(0, TB // 2, blk2, 0)

        dbase = wid * (B_ALIGN // NW)
        pltpu.sync_copy(di_ref.at[pl.ds(dbase, B_ALIGN // NW)],
                        tib.at[pl.ds(0, B_ALIGN // NW)])
        pltpu.sync_copy(g_ref.at[tib.at[pl.ds(0, B_ALIGN // NW)]], gh[0])
        pltpu.sync_copy(gh[0], dout_ref.at[pl.ds(dbase, B_ALIGN // NW)])

        rbase = wid * (B_REL // NW)
        pltpu.sync_copy(ri_ref.at[pl.ds(rbase, B_REL // NW)], rlx)
        pltpu.sync_copy(r_ref.at[rlx], rbb)
        pltpu.sync_copy(rbb, rout_ref.at[pl.ds(rbase, B_REL // NW)])

    return body(g, rnorm, ti, d_idx, rl_idx)


def _pad_idx(x, n):
    x = x.astype(jnp.int32)
    return jnp.concatenate([x, jnp.zeros((n - x.shape[0],), jnp.int32)])


def _gat_graph(x, edge_index, wcats, u1s, u2s):
    src = edge_index[0].astype(jnp.int32)
    dst = edge_index[1].astype(jnp.int32)
    loop = jnp.arange(N, dtype=jnp.int32)
    padv = jnp.full((EPAD - ETOT,), N, jnp.int32)
    src_all = jnp.concatenate([src, loop, padv]).reshape(NW * EB, KE)
    dst_all = jnp.concatenate([dst, loop, padv]).reshape(NW * EB, KE)
    pe = jnp.stack([src_all, dst_all], axis=1).reshape(NW * EB * 2 * KE)

    xp = jnp.concatenate([x, jnp.zeros((NP - N, DIM), jnp.float32)])
    wh, s1, s2 = _proj(xp, wcats[0], u1s[0], u2s[0])
    acc = _edge_pass(pe, wh, s1, s2)
    wh, s1, s2 = _combine_elu_proj(acc, wcats[1], u1s[1], u2s[1])
    acc = _edge_pass(pe, wh, s1, s2)
    return _combine_norm(acc)


def kernel(sr_data, tg_data, sr_rel_data, tg_rel_data, triples_sr_h, triples_sr_t, triples_sr_r, triples_tg_h, triples_tg_t, triples_tg_r, edge_index_sr, edge_index_tg, ent_emb_sr, ent_emb_tg, rel_emb_sr, rel_emb_tg, gat_W, gat_a_src, gat_a_dst):
    wcats, u1s, u2s = [], [], []
    pidx = np.array([(j % NHEADS) * DH + j // NHEADS for j in range(DIM)])
    for l in range(NUM_LAYER):
        w = gat_W[l]
        wcat = w.transpose(1, 2, 0).reshape(DIM, DIM)
        us = jnp.einsum('hdk,hk->dh', w, gat_a_src[l])
        ud = jnp.einsum('hdk,hk->dh', w, gat_a_dst[l])
        u1 = jnp.concatenate([us, us], axis=1)
        u2 = jnp.concatenate([ud, ud], axis=1)
        if l > 0:
            wcat, u1, u2 = wcat[pidx], u1[pidx], u2[pidx]
        wcats.append(wcat)
        u1s.append(u1)
        u2s.append(u2)

    g_sr = _gat_graph(ent_emb_sr, edge_index_sr, wcats, u1s, u2s)
    g_tg = _gat_graph(ent_emb_tg, edge_index_tg, wcats, u1s, u2s)

    rels = _relnorm(jnp.concatenate([rel_emb_sr, rel_emb_tg]))
    r_sr, r_tg = rels[:R], rels[R:]

    def pack_ti(h, t, r):
        arrs = [_pad_idx(x, TP).reshape(NW, TB, K) for x in (h, t, r)]
        return jnp.stack(arrs, axis=2).reshape(TP * 3)

    tv_sr, sr_data_repre, sr_rel_repre = _transe_gather(
        g_sr, r_sr, pack_ti(triples_sr_h, triples_sr_t, triples_sr_r),
        sr_data.astype(jnp.int32), sr_rel_data.astype(jnp.int32))
    tv_tg, tg_data_repre, tg_rel_repre = _transe_gather(
        g_tg, r_tg, pack_ti(triples_tg_h, triples_tg_t, triples_tg_r),
        tg_data.astype(jnp.int32), tg_rel_data.astype(jnp.int32))

    transe_tv = jnp.concatenate([tv_sr[:T], tv_tg[:T]])
    return (sr_data_repre, tg_data_repre, sr_rel_repre, tg_rel_repre, transe_tv)


# speedup vs baseline: 1.0091x; 1.0091x over previous
import functools

import jax
import jax.numpy as jnp
import numpy as np
from jax import lax
from jax.experimental import pallas as pl
from jax.experimental.pallas import tpu as pltpu
from jax.experimental.pallas import tpu_sc as plsc

N = 10000
DIM = 128
NHEADS = 8
DH = DIM // NHEADS
NUM_LAYER = 2
E = 320000
R = 1000
B_ALIGN = 4096
B_REL = 512
T = 100000
ALPHA = 0.2

NC = 2
NS = 16
NW = NC * NS
K = 128

NP = 10112
ACC_W = 144
ETOT = E + N
KE = 64
EB = 162
EPW = EB * KE
EPAD = EPW * NW
TB = 26
TPW = TB * K
TP = TPW * NW

_INV3SQ = float(1.0 / (3.0 * np.sqrt(float(DIM))))


def _lane_bcast(v, idx16):
    return lax.gather(
        v, idx16[:, None],
        lax.GatherDimensionNumbers(offset_dims=(), collapsed_slice_dims=(0,),
                                   start_index_map=(0,)),
        (1,), mode=lax.GatherScatterMode.PROMISE_IN_BOUNDS)


def _proj_body(x_ref, wcat_ref, u1_ref, u2_ref, wh_ref, s1_ref, s2_ref):
    x = x_ref[...]
    wh_ref[...] = jnp.dot(x, wcat_ref[...], preferred_element_type=jnp.float32)
    s1_ref[...] = jnp.dot(x, u1_ref[...], preferred_element_type=jnp.float32)
    s2_ref[...] = jnp.dot(x, u2_ref[...], preferred_element_type=jnp.float32)


def _proj(x, wcat, u1, u2):
    grid = 4
    b = NP // grid
    return pl.pallas_call(
        _proj_body,
        grid=(grid,),
        in_specs=[
            pl.BlockSpec((b, DIM), lambda i: (i, 0)),
            pl.BlockSpec((DIM, DIM), lambda i: (0, 0)),
            pl.BlockSpec((DIM, 16), lambda i: (0, 0)),
            pl.BlockSpec((DIM, 16), lambda i: (0, 0)),
        ],
        out_specs=[
            pl.BlockSpec((b, DIM), lambda i: (i, 0)),
            pl.BlockSpec((b, 16), lambda i: (i, 0)),
            pl.BlockSpec((b, 16), lambda i: (i, 0)),
        ],
        out_shape=[
            jax.ShapeDtypeStruct((NP, DIM), jnp.float32),
            jax.ShapeDtypeStruct((NP, 16), jnp.float32),
            jax.ShapeDtypeStruct((NP, 16), jnp.float32),
        ],
    )(x, wcat, u1, u2)


def _rep8():
    row = lax.broadcasted_iota(jnp.int32, (NHEADS, DIM), 0)
    lane = lax.broadcasted_iota(jnp.int32, (NHEADS, DIM), 1) % NHEADS
    return (row == lane).astype(jnp.float32)


def _combine_elu_proj_body(acc_ref, wcat_ref, u1_ref, u2_ref,
                           wh_ref, s1_ref, s2_ref):
    a = acc_ref[0] + acc_ref[1]
    drep = jnp.dot(a[:, DIM:DIM + NHEADS], _rep8(),
                   preferred_element_type=jnp.float32)
    x = a[:, :DIM] / (drep + 1e-16)
    x = jnp.where(x > 0, x, jnp.exp(x) - 1.0)
    wh_ref[...] = jnp.dot(x, wcat_ref[...], preferred_element_type=jnp.float32)
    s1_ref[...] = jnp.dot(x, u1_ref[...], preferred_element_type=jnp.float32)
    s2_ref[...] = jnp.dot(x, u2_ref[...], preferred_element_type=jnp.float32)


def _combine_elu_proj(acc, wcat, u1, u2):
    grid = 4
    b = NP // grid
    return pl.pallas_call(
        _combine_elu_proj_body,
        grid=(grid,),
        in_specs=[
            pl.BlockSpec((2, b, ACC_W), lambda i: (0, i, 0)),
            pl.BlockSpec((DIM, DIM), lambda i: (0, 0)),
            pl.BlockSpec((DIM, 16), lambda i: (0, 0)),
            pl.BlockSpec((DIM, 16), lambda i: (0, 0)),
        ],
        out_specs=[
            pl.BlockSpec((b, DIM), lambda i: (i, 0)),
            pl.BlockSpec((b, 16), lambda i: (i, 0)),
            pl.BlockSpec((b, 16), lambda i: (i, 0)),
        ],
        out_shape=[
            jax.ShapeDtypeStruct((NP, DIM), jnp.float32),
            jax.ShapeDtypeStruct((NP, 16), jnp.float32),
            jax.ShapeDtypeStruct((NP, 16), jnp.float32),
        ],
    )(acc, wcat, u1, u2)


def _combine_norm_body(acc_ref, g_ref):
    a = acc_ref[0] + acc_ref[1]
    drep = jnp.dot(a[:, DIM:DIM + NHEADS], _rep8(),
                   preferred_element_type=jnp.float32)
    g = a[:, :DIM] / (drep + 1e-16)
    rowj = lax.broadcasted_iota(jnp.int32, (DIM, DIM), 0)
    colo = lax.broadcasted_iota(jnp.int32, (DIM, DIM), 1)
    pm = (colo == (rowj % NHEADS) * DH + rowj // NHEADS).astype(jnp.float32)
    g = jnp.dot(g, pm, preferred_element_type=jnp.float32)
    nrm = jnp.sqrt(jnp.sum(g * g, axis=1, keepdims=True))
    g_ref[...] = g / (nrm + 1e-12)


def _combine_norm(acc):
    grid = 4
    b = NP // grid
    return pl.pallas_call(
        _combine_norm_body,
        grid=(grid,),
        in_specs=[pl.BlockSpec((2, b, ACC_W), lambda i: (0, i, 0))],
        out_specs=pl.BlockSpec((b, DIM), lambda i: (i, 0)),
        out_shape=jax.ShapeDtypeStruct((NP, DIM), jnp.float32),
    )(acc)


def _relnorm_body(x_ref, o_ref):
    x = x_ref[...]
    nrm = jnp.sqrt(jnp.sum(x * x, axis=1, keepdims=True))
    o_ref[...] = x / (nrm + 1e-12)


def _relnorm(x):
    return pl.pallas_call(
        _relnorm_body,
        out_shape=jax.ShapeDtypeStruct(x.shape, jnp.float32),
    )(x)


def _edge_pass(pe, wh, s1, s2):
    mesh = plsc.VectorSubcoreMesh(core_axis_name="c", subcore_axis_name="s")

    @functools.partial(
        pl.kernel,
        out_type=jax.ShapeDtypeStruct((NC, NP, ACC_W), jnp.float32),
        mesh=mesh,
        scratch_types=[
            [pltpu.VMEM((2 * KE,), jnp.int32)] * 2,
            [pltpu.VMEM((KE,), jnp.int32)] * 2,
            [pltpu.VMEM((KE, 16), jnp.float32)] * 2,
            [pltpu.VMEM((KE, 16), jnp.float32)] * 2,
            [pltpu.VMEM((KE, DIM), jnp.float32)] * 2,
            [pltpu.VMEM((KE, ACC_W), jnp.float32)] * 2,
            [pltpu.SemaphoreType.DMA] * 2,
            [pltpu.SemaphoreType.DMA] * 2,
            [pltpu.SemaphoreType.DMA] * 2,
            pltpu.VMEM_SHARED((NP, ACC_W), jnp.float32),
        ],
        compiler_params=pltpu.CompilerParams(use_tc_tiling_on_sc=False,
                                             needs_layout_passes=False),
    )
    def body(pe_ref, wh_ref, s1_ref, s2_ref, out_ref,
             ibuf, sidx, ssb, sdb, gwb, obb, isem, gsem, ssem, acc):
        c = lax.axis_index("c")
        s = lax.axis_index("s")
        wid = c * NS + s

        def zrow(e, carry):
            for j in range(ACC_W // 16):
                obb[0][e, pl.ds(j * 16, 16)] = jnp.zeros((16,), jnp.float32)
            return carry

        lax.fori_loop(0, KE, zrow, 0)

        row0 = s * (NP // NS)
        for off in range(0, NP // NS, KE):
            nr = min(KE, NP // NS - off)
            pltpu.sync_copy(obb[0].at[pl.ds(0, nr)],
                            acc.at[pl.ds(row0 + off, nr)])
        plsc.subcore_barrier()

        lanes = lax.broadcasted_iota(jnp.int32, (16,), 0)
        wstart = wid * EB * 2 * KE

        def idx_fetch(b, p):
            pltpu.async_copy(pe_ref.at[pl.ds(wstart + b * 2 * KE, 2 * KE)],
                             ibuf[p], isem[p])

        def idx_wait(p):
            pltpu.make_async_copy(pe_ref.at[pl.ds(0, 2 * KE)], ibuf[p],
                                  isem[p]).wait()

        def fire_gathers(p):
            pltpu.async_copy(s1_ref.at[ibuf[p].at[pl.ds(0, KE)]],
                             ssb[p], gsem[p])
            pltpu.async_copy(s2_ref.at[ibuf[p].at[pl.ds(KE, KE)]],
                             sdb[p], gsem[p])
            pltpu.async_copy(wh_ref.at[ibuf[p].at[pl.ds(0, KE)]],
                             gwb[p], gsem[p])

        def wait_gathers(p):
            pltpu.make_async_copy(s1_ref.at[ibuf[p].at[pl.ds(0, KE)]],
                                  ssb[p], gsem[p]).wait()
            pltpu.make_async_copy(s2_ref.at[ibuf[p].at[pl.ds(KE, KE)]],
                                  sdb[p], gsem[p]).wait()
            pltpu.make_async_copy(wh_ref.at[ibuf[p].at[pl.ds(0, KE)]],
                                  gwb[p], gsem[p]).wait()

        pltpu.sync_copy(pe_ref.at[pl.ds(wstart, 2 * KE)], ibuf[0])
        fire_gathers(0)
        idx_fetch(1, 1)

        def blk2(i, carry):
            for p in range(2):
                q = 1 - p
                b = 2 * i + p
                wait_gathers(p)

                @pl.when(b >= 2)
                def _():
                    pltpu.make_async_copy(
                        obb[p], acc.at[sidx[p]], ssem[p]).wait()

                for e in range(KE):
                    t = ssb[p][e, :] + sdb[p][e, :]
                    w = jnp.exp(jnp.where(t > 0, t, ALPHA * t))
                    obb[p][e, pl.ds(DIM, 16)] = jnp.where(
                        lanes < NHEADS, w, 0.0)
                    for ch in range(DIM // 16):
                        obb[p][e, pl.ds(ch * 16, 16)] = (
                            gwb[p][e, pl.ds(ch * 16, 16)] * w)

                for j in range(KE // 16):
                    sidx[p][pl.ds(j * 16, 16)] = ibuf[p][pl.ds(KE + j * 16,
                                                               16)]
                pltpu.async_copy(obb[p], acc.at[sidx[p]], ssem[p], add=True)

                @pl.when(b + 2 < EB)
                def _():
                    idx_fetch(b + 2, p)

                @pl.when(b + 1 < EB)
                def _():
                    idx_wait(q)
                    fire_gathers(q)
            return carry

        lax.fori_loop(0, EB // 2, blk2, 0)
        for p in range(2):
            pltpu.make_async_copy(obb[p], acc.at[sidx[p]], ssem[p]).wait()
        plsc.subcore_barrier()
        nr = NP // NS
        pltpu.sync_copy(acc.at[pl.ds(row0, nr)], out_ref.at[c, pl.ds(row0, nr)])

    return body(pe, wh, s1, s2)


def _transe_gather(g, rnorm, ti, d_idx, rl_idx):
    mesh = plsc.VectorSubcoreMesh(core_axis_name="c", subcore_axis_name="s")

    @functools.partial(
        pl.kernel,
        out_type=(
            jax.ShapeDtypeStruct((TP,), jnp.float32),
            jax.ShapeDtypeStruct((B_ALIGN, DIM), jnp.float32),
            jax.ShapeDtypeStruct((B_REL, DIM), jnp.float32),
        ),
        mesh=mesh,
        scratch_types=[
            pltpu.VMEM((TB * 3 * K,), jnp.int32),
            [pltpu.VMEM((K, DIM), jnp.float32)] * 2,
            [pltpu.VMEM((K, DIM), jnp.float32)] * 2,
            [pltpu.VMEM((K, DIM), jnp.float32)] * 2,
            [pltpu.SemaphoreType.DMA] * 2,
            pltpu.VMEM((K,), jnp.float32),
            pltpu.VMEM((16,), jnp.int32),
            pltpu.VMEM((16, DIM), jnp.float32),
        ],
        compiler_params=pltpu.CompilerParams(use_tc_tiling_on_sc=False,
                                             needs_layout_passes=False),
    )
    def body(g_ref, r_ref, ti_ref, di_ref, ri_ref,
             tv_ref, dout_ref, rout_ref,
             tib, gh, gt, gr, gsem, tvb, rlx, rbb):
        c = lax.axis_index("c")
        s = lax.axis_index("s")
        wid = c * NS + s

        pltpu.sync_copy(ti_ref.at[pl.ds(wid * TB * 3 * K, TB * 3 * K)], tib)

        def fire(b, p):
            off = pl.multiple_of(b * 3 * K, 128)
            pltpu.async_copy(g_ref.at[tib.at[pl.ds(off, K)]], gh[p], gsem[p])
            pltpu.async_copy(g_ref.at[tib.at[pl.ds(off + K, K)]],
                             gt[p], gsem[p])
            pltpu.async_copy(r_ref.at[tib.at[pl.ds(off + 2 * K, K)]],
                             gr[p], gsem[p])

        def wait_g(p):
            pltpu.make_async_copy(g_ref.at[tib.at[pl.ds(0, K)]],
                                  gh[p], gsem[p]).wait()
            pltpu.make_async_copy(g_ref.at[tib.at[pl.ds(0, K)]],
                                  gt[p], gsem[p]).wait()
            pltpu.make_async_copy(r_ref.at[tib.at[pl.ds(0, K)]],
                                  gr[p], gsem[p]).wait()

        fire(0, 0)
        lanes = lax.broadcasted_iota(jnp.int32, (16,), 0)

        def blk2(i, carry):
            for p in range(2):
                q = 1 - p
                b = 2 * i + p
                wait_g(p)

                @pl.when(b + 1 < TB)
                def _():
                    fire(b + 1, q)

                lane15 = jnp.full((16,), 15, jnp.int32)

                def tri16(g16, ecarry):
                    res = jnp.zeros((16,), jnp.float32)
                    for j in range(16):
                        e = g16 * 16 + j
                        acc = jnp.zeros((16,), jnp.float32)
                        for ch in range(DIM // 16):
                            sl = pl.ds(ch * 16, 16)
                            acc = acc + jnp.abs(
                                gh[p][e, sl] + gr[p][e, sl] - gt[p][e, sl])
                        bsum = _lane_bcast(plsc.cumsum(acc), lane15)
                        res = jnp.where(lanes == j,
                                        1.0 - bsum * _INV3SQ, res)
                    tvb[pl.ds(g16 * 16, 16)] = res
                    return ecarry

                lax.fori_loop(0, K // 16, tri16, 0)
                pltpu.sync_copy(tvb, tv_ref.at[pl.ds(wid * TPW + b * K, K)])
            return carry

        lax.fori_loop(0, TB // 2, blk2, 0)

        dbase = wid * (B_ALIGN // NW)
        pltpu.sync_copy(di_ref.at[pl.ds(dbase, B_ALIGN // NW)],
                        tib.at[pl.ds(0, B_ALIGN // NW)])
        pltpu.sync_copy(g_ref.at[tib.at[pl.ds(0, B_ALIGN // NW)]], gh[0])
        pltpu.sync_copy(gh[0], dout_ref.at[pl.ds(dbase, B_ALIGN // NW)])

        rbase = wid * (B_REL // NW)
        pltpu.sync_copy(ri_ref.at[pl.ds(rbase, B_REL // NW)], rlx)
        pltpu.sync_copy(r_ref.at[rlx], rbb)
        pltpu.sync_copy(rbb, rout_ref.at[pl.ds(rbase, B_REL // NW)])

    return body(g, rnorm, ti, d_idx, rl_idx)


def _pad_idx(x, n):
    x = x.astype(jnp.int32)
    return jnp.concatenate([x, jnp.zeros((n - x.shape[0],), jnp.int32)])


def _gat_graph(x, edge_index, wcats, u1s, u2s):
    src = edge_index[0].astype(jnp.int32)
    dst = edge_index[1].astype(jnp.int32)
    loop = jnp.arange(N, dtype=jnp.int32)
    padv = jnp.full((EPAD - ETOT,), N, jnp.int32)
    src_all = jnp.concatenate([src, loop, padv]).reshape(NW * EB, KE)
    dst_all = jnp.concatenate([dst, loop, padv]).reshape(NW * EB, KE)
    pe = jnp.stack([src_all, dst_all], axis=1).reshape(NW * EB * 2 * KE)

    xp = jnp.concatenate([x, jnp.zeros((NP - N, DIM), jnp.float32)])
    wh, s1, s2 = _proj(xp, wcats[0], u1s[0], u2s[0])
    acc = _edge_pass(pe, wh, s1, s2)
    wh, s1, s2 = _combine_elu_proj(acc, wcats[1], u1s[1], u2s[1])
    acc = _edge_pass(pe, wh, s1, s2)
    return _combine_norm(acc)


def kernel(sr_data, tg_data, sr_rel_data, tg_rel_data, triples_sr_h, triples_sr_t, triples_sr_r, triples_tg_h, triples_tg_t, triples_tg_r, edge_index_sr, edge_index_tg, ent_emb_sr, ent_emb_tg, rel_emb_sr, rel_emb_tg, gat_W, gat_a_src, gat_a_dst):
    wcats, u1s, u2s = [], [], []
    pidx = np.array([(j % NHEADS) * DH + j // NHEADS for j in range(DIM)])
    for l in range(NUM_LAYER):
        w = gat_W[l]
        wcat = w.transpose(1, 2, 0).reshape(DIM, DIM)
        us = jnp.einsum('hdk,hk->dh', w, gat_a_src[l])
        ud = jnp.einsum('hdk,hk->dh', w, gat_a_dst[l])
        u1 = jnp.concatenate([us, us], axis=1)
        u2 = jnp.concatenate([ud, ud], axis=1)
        if l > 0:
            wcat, u1, u2 = wcat[pidx], u1[pidx], u2[pidx]
        wcats.append(wcat)
        u1s.append(u1)
        u2s.append(u2)

    g_sr = _gat_graph(ent_emb_sr, edge_index_sr, wcats, u1s, u2s)
    g_tg = _gat_graph(ent_emb_tg, edge_index_tg, wcats, u1s, u2s)

    rels = _relnorm(jnp.concatenate([rel_emb_sr, rel_emb_tg]))
    r_sr, r_tg = rels[:R], rels[R:]

    def pack_ti(h, t, r):
        arrs = [_pad_idx(x, TP).reshape(NW, TB, K) for x in (h, t, r)]
        return jnp.stack(arrs, axis=2).reshape(TP * 3)

    tv_sr, sr_data_repre, sr_rel_repre = _transe_gather(
        g_sr, r_sr, pack_ti(triples_sr_h, triples_sr_t, triples_sr_r),
        sr_data.astype(jnp.int32), sr_rel_data.astype(jnp.int32))
    tv_tg, tg_data_repre, tg_rel_repre = _transe_gather(
        g_tg, r_tg, pack_ti(triples_tg_h, triples_tg_t, triples_tg_r),
        tg_data.astype(jnp.int32), tg_rel_data.astype(jnp.int32))

    transe_tv = jnp.concatenate([tv_sr[:T], tv_tg[:T]])
    return (sr_data_repre, tg_data_repre, sr_rel_repre, tg_rel_repre, transe_tv)


# speedup vs baseline: 1.3801x; 1.3677x over previous
import functools

import jax
import jax.numpy as jnp
import numpy as np
from jax import lax
from jax.experimental import pallas as pl
from jax.experimental.pallas import tpu as pltpu
from jax.experimental.pallas import tpu_sc as plsc

N = 10000
DIM = 128
NHEADS = 8
DH = DIM // NHEADS
NUM_LAYER = 2
E = 320000
R = 1000
B_ALIGN = 4096
B_REL = 512
T = 100000
ALPHA = 0.2

NC = 2
NS = 16
NW = NC * NS
K = 128

NP = 10112
ACC_W = 144
ETOT = E + N
KE = 64
EB = 162
EPW = EB * KE
EPAD = EPW * NW
TB = 26
TPW = TB * K
TP = TPW * NW

_INV3SQ = float(1.0 / (3.0 * np.sqrt(float(DIM))))


def _lane_bcast(v, idx16):
    return lax.gather(
        v, idx16[:, None],
        lax.GatherDimensionNumbers(offset_dims=(), collapsed_slice_dims=(0,),
                                   start_index_map=(0,)),
        (1,), mode=lax.GatherScatterMode.PROMISE_IN_BOUNDS)


def _proj_body(x_ref, wcat_ref, u1_ref, u2_ref, wh_ref, s1_ref, s2_ref):
    x = x_ref[...]
    wh_ref[...] = jnp.dot(x, wcat_ref[...], preferred_element_type=jnp.float32)
    s1_ref[...] = jnp.dot(x, u1_ref[...], preferred_element_type=jnp.float32)
    s2_ref[...] = jnp.dot(x, u2_ref[...], preferred_element_type=jnp.float32)


def _proj(x, wcat, u1, u2):
    grid = 4
    b = NP // grid
    return pl.pallas_call(
        _proj_body,
        grid=(grid,),
        in_specs=[
            pl.BlockSpec((b, DIM), lambda i: (i, 0)),
            pl.BlockSpec((DIM, DIM), lambda i: (0, 0)),
            pl.BlockSpec((DIM, 16), lambda i: (0, 0)),
            pl.BlockSpec((DIM, 16), lambda i: (0, 0)),
        ],
        out_specs=[
            pl.BlockSpec((b, DIM), lambda i: (i, 0)),
            pl.BlockSpec((b, 16), lambda i: (i, 0)),
            pl.BlockSpec((b, 16), lambda i: (i, 0)),
        ],
        out_shape=[
            jax.ShapeDtypeStruct((NP, DIM), jnp.float32),
            jax.ShapeDtypeStruct((NP, 16), jnp.float32),
            jax.ShapeDtypeStruct((NP, 16), jnp.float32),
        ],
    )(x, wcat, u1, u2)


def _rep8():
    row = lax.broadcasted_iota(jnp.int32, (NHEADS, DIM), 0)
    lane = lax.broadcasted_iota(jnp.int32, (NHEADS, DIM), 1) % NHEADS
    return (row == lane).astype(jnp.float32)


def _combine_elu_proj_body(acc_ref, wcat_ref, u1_ref, u2_ref,
                           wh_ref, s1_ref, s2_ref):
    a = acc_ref[0] + acc_ref[1]
    drep = jnp.dot(a[:, DIM:DIM + NHEADS], _rep8(),
                   preferred_element_type=jnp.float32)
    x = a[:, :DIM] / (drep + 1e-16)
    x = jnp.where(x > 0, x, jnp.exp(x) - 1.0)
    wh_ref[...] = jnp.dot(x, wcat_ref[...], preferred_element_type=jnp.float32)
    s1_ref[...] = jnp.dot(x, u1_ref[...], preferred_element_type=jnp.float32)
    s2_ref[...] = jnp.dot(x, u2_ref[...], preferred_element_type=jnp.float32)


def _combine_elu_proj(acc, wcat, u1, u2):
    grid = 4
    b = NP // grid
    return pl.pallas_call(
        _combine_elu_proj_body,
        grid=(grid,),
        in_specs=[
            pl.BlockSpec((2, b, ACC_W), lambda i: (0, i, 0)),
            pl.BlockSpec((DIM, DIM), lambda i: (0, 0)),
            pl.BlockSpec((DIM, 16), lambda i: (0, 0)),
            pl.BlockSpec((DIM, 16), lambda i: (0, 0)),
        ],
        out_specs=[
            pl.BlockSpec((b, DIM), lambda i: (i, 0)),
            pl.BlockSpec((b, 16), lambda i: (i, 0)),
            pl.BlockSpec((b, 16), lambda i: (i, 0)),
        ],
        out_shape=[
            jax.ShapeDtypeStruct((NP, DIM), jnp.float32),
            jax.ShapeDtypeStruct((NP, 16), jnp.float32),
            jax.ShapeDtypeStruct((NP, 16), jnp.float32),
        ],
    )(acc, wcat, u1, u2)


def _combine_norm_body(acc_ref, g_ref):
    a = acc_ref[0] + acc_ref[1]
    drep = jnp.dot(a[:, DIM:DIM + NHEADS], _rep8(),
                   preferred_element_type=jnp.float32)
    g = a[:, :DIM] / (drep + 1e-16)
    rowj = lax.broadcasted_iota(jnp.int32, (DIM, DIM), 0)
    colo = lax.broadcasted_iota(jnp.int32, (DIM, DIM), 1)
    pm = (colo == (rowj % NHEADS) * DH + rowj // NHEADS).astype(jnp.float32)
    g = jnp.dot(g, pm, preferred_element_type=jnp.float32)
    nrm = jnp.sqrt(jnp.sum(g * g, axis=1, keepdims=True))
    g_ref[...] = g / (nrm + 1e-12)


def _combine_norm(acc):
    grid = 4
    b = NP // grid
    return pl.pallas_call(
        _combine_norm_body,
        grid=(grid,),
        in_specs=[pl.BlockSpec((2, b, ACC_W), lambda i: (0, i, 0))],
        out_specs=pl.BlockSpec((b, DIM), lambda i: (i, 0)),
        out_shape=jax.ShapeDtypeStruct((NP, DIM), jnp.float32),
    )(acc)


def _relnorm_body(x_ref, o_ref):
    x = x_ref[...]
    nrm = jnp.sqrt(jnp.sum(x * x, axis=1, keepdims=True))
    o_ref[...] = x / (nrm + 1e-12)


def _relnorm(x):
    return pl.pallas_call(
        _relnorm_body,
        out_shape=jax.ShapeDtypeStruct(x.shape, jnp.float32),
    )(x)


def _edge_pass(pe, wh, s1, s2):
    mesh = plsc.VectorSubcoreMesh(core_axis_name="c", subcore_axis_name="s")

    @functools.partial(
        pl.kernel,
        out_type=jax.ShapeDtypeStruct((NC, NP, ACC_W), jnp.float32),
        mesh=mesh,
        scratch_types=[
            [pltpu.VMEM((2 * KE,), jnp.int32)] * 2,
            [pltpu.VMEM((KE,), jnp.int32)] * 2,
            [pltpu.VMEM((KE, 16), jnp.float32)] * 2,
            [pltpu.VMEM((KE, 16), jnp.float32)] * 2,
            [pltpu.VMEM((KE, DIM), jnp.float32)] * 2,
            [pltpu.VMEM((KE, ACC_W), jnp.float32)] * 2,
            [pltpu.SemaphoreType.DMA] * 2,
            [pltpu.SemaphoreType.DMA] * 2,
            [pltpu.SemaphoreType.DMA] * 2,
            pltpu.VMEM_SHARED((NP, ACC_W), jnp.float32),
        ],
        compiler_params=pltpu.CompilerParams(use_tc_tiling_on_sc=False,
                                             needs_layout_passes=False),
    )
    def body(pe_ref, wh_ref, s1_ref, s2_ref, out_ref,
             ibuf, sidx, ssb, sdb, gwb, obb, isem, gsem, ssem, acc):
        c = lax.axis_index("c")
        s = lax.axis_index("s")
        wid = c * NS + s

        def zrow(e, carry):
            for j in range(ACC_W // 16):
                obb[0][e, pl.ds(j * 16, 16)] = jnp.zeros((16,), jnp.float32)
            return carry

        lax.fori_loop(0, KE, zrow, 0)

        row0 = s * (NP // NS)
        for off in range(0, NP // NS, KE):
            nr = min(KE, NP // NS - off)
            pltpu.sync_copy(obb[0].at[pl.ds(0, nr)],
                            acc.at[pl.ds(row0 + off, nr)])
        plsc.subcore_barrier()

        lanes = lax.broadcasted_iota(jnp.int32, (16,), 0)
        wstart = wid * EB * 2 * KE

        def idx_fetch(b, p):
            pltpu.async_copy(pe_ref.at[pl.ds(wstart + b * 2 * KE, 2 * KE)],
                             ibuf[p], isem[p])

        def idx_wait(p):
            pltpu.make_async_copy(pe_ref.at[pl.ds(0, 2 * KE)], ibuf[p],
                                  isem[p]).wait()

        def fire_gathers(p):
            pltpu.async_copy(s1_ref.at[ibuf[p].at[pl.ds(0, KE)]],
                             ssb[p], gsem[p])
            pltpu.async_copy(s2_ref.at[ibuf[p].at[pl.ds(KE, KE)]],
                             sdb[p], gsem[p])
            pltpu.async_copy(wh_ref.at[ibuf[p].at[pl.ds(0, KE)]],
                             gwb[p], gsem[p])

        def wait_gathers(p):
            pltpu.make_async_copy(s1_ref.at[ibuf[p].at[pl.ds(0, KE)]],
                                  ssb[p], gsem[p]).wait()
            pltpu.make_async_copy(s2_ref.at[ibuf[p].at[pl.ds(KE, KE)]],
                                  sdb[p], gsem[p]).wait()
            pltpu.make_async_copy(wh_ref.at[ibuf[p].at[pl.ds(0, KE)]],
                                  gwb[p], gsem[p]).wait()

        pltpu.sync_copy(pe_ref.at[pl.ds(wstart, 2 * KE)], ibuf[0])
        fire_gathers(0)
        idx_fetch(1, 1)

        def blk2(i, carry):
            for p in range(2):
                q = 1 - p
                b = 2 * i + p
                wait_gathers(p)

                @pl.when(b >= 2)
                def _():
                    pltpu.make_async_copy(
                        obb[p], acc.at[sidx[p]], ssem[p]).wait()

                for e in range(KE):
                    t = ssb[p][e, :] + sdb[p][e, :]
                    w = jnp.exp(jnp.where(t > 0, t, ALPHA * t))
                    obb[p][e, pl.ds(DIM, 16)] = jnp.where(
                        lanes < NHEADS, w, 0.0)
                    for ch in range(DIM // 16):
                        obb[p][e, pl.ds(ch * 16, 16)] = (
                            gwb[p][e, pl.ds(ch * 16, 16)] * w)

                for j in range(KE // 16):
                    sidx[p][pl.ds(j * 16, 16)] = ibuf[p][pl.ds(KE + j * 16,
                                                               16)]
                pltpu.async_copy(obb[p], acc.at[sidx[p]], ssem[p], add=True)

                @pl.when(b + 2 < EB)
                def _():
                    idx_fetch(b + 2, p)

                @pl.when(b + 1 < EB)
                def _():
                    idx_wait(q)
                    fire_gathers(q)
            return carry

        lax.fori_loop(0, EB // 2, blk2, 0)
        for p in range(2):
            pltpu.make_async_copy(obb[p], acc.at[sidx[p]], ssem[p]).wait()
        plsc.subcore_barrier()
        nr = NP // NS
        pltpu.sync_copy(acc.at[pl.ds(row0, nr)], out_ref.at[c, pl.ds(row0, nr)])

    return body(pe, wh, s1, s2)


def _transe_gather(g, g_bf, rnorm, r_bf, ti, d_idx, rl_idx):
    mesh = plsc.VectorSubcoreMesh(core_axis_name="c", subcore_axis_name="s")

    @functools.partial(
        pl.kernel,
        out_type=(
            jax.ShapeDtypeStruct((TP,), jnp.float32),
            jax.ShapeDtypeStruct((B_ALIGN, DIM), jnp.float32),
            jax.ShapeDtypeStruct((B_REL, DIM), jnp.float32),
        ),
        mesh=mesh,
        scratch_types=[
            pltpu.VMEM((TB * 3 * K,), jnp.int32),
            [pltpu.VMEM((K, DIM), jnp.bfloat16)] * 2,
            [pltpu.VMEM((K, DIM), jnp.bfloat16)] * 2,
            [pltpu.VMEM((K, DIM), jnp.bfloat16)] * 2,
            [pltpu.SemaphoreType.DMA] * 2,
            pltpu.VMEM((K,), jnp.float32),
            pltpu.VMEM((16,), jnp.int32),
            pltpu.VMEM((16, DIM), jnp.float32),
            pltpu.VMEM((K, DIM), jnp.float32),
        ],
        compiler_params=pltpu.CompilerParams(use_tc_tiling_on_sc=False,
                                             needs_layout_passes=False),
    )
    def body(g_ref, gbf_ref, r_ref, rbf_ref, ti_ref, di_ref, ri_ref,
             tv_ref, dout_ref, rout_ref,
             tib, gh, gt, gr, gsem, tvb, rlx, rbb, fbb):
        c = lax.axis_index("c")
        s = lax.axis_index("s")
        wid = c * NS + s

        pltpu.sync_copy(ti_ref.at[pl.ds(wid * TB * 3 * K, TB * 3 * K)], tib)

        def fire(b, p):
            off = pl.multiple_of(b * 3 * K, 128)
            pltpu.async_copy(gbf_ref.at[tib.at[pl.ds(off, K)]], gh[p], gsem[p])
            pltpu.async_copy(gbf_ref.at[tib.at[pl.ds(off + K, K)]],
                             gt[p], gsem[p])
            pltpu.async_copy(rbf_ref.at[tib.at[pl.ds(off + 2 * K, K)]],
                             gr[p], gsem[p])

        def wait_g(p):
            pltpu.make_async_copy(gbf_ref.at[tib.at[pl.ds(0, K)]],
                                  gh[p], gsem[p]).wait()
            pltpu.make_async_copy(gbf_ref.at[tib.at[pl.ds(0, K)]],
                                  gt[p], gsem[p]).wait()
            pltpu.make_async_copy(rbf_ref.at[tib.at[pl.ds(0, K)]],
                                  gr[p], gsem[p]).wait()

        fire(0, 0)
        lanes = lax.broadcasted_iota(jnp.int32, (16,), 0)

        def blk2(i, carry):
            for p in range(2):
                q = 1 - p
                b = 2 * i + p
                wait_g(p)

                @pl.when(b + 1 < TB)
                def _():
                    fire(b + 1, q)

                lane15 = jnp.full((16,), 15, jnp.int32)

                def tri16(g16, ecarry):
                    res = jnp.zeros((16,), jnp.float32)
                    for j in range(16):
                        e = g16 * 16 + j
                        acc = jnp.zeros((16,), jnp.float32)
                        for ch in range(DIM // 32):
                            sl = pl.ds(ch * 32, 32)
                            sab = jnp.abs(gh[p][e, sl] + gr[p][e, sl]
                                          - gt[p][e, sl])
                            lo, hi = plsc.unpack(
                                sab, format=plsc.PackFormat.INTERLEAVED)
                            acc = acc + lo + hi
                        bsum = _lane_bcast(plsc.cumsum(acc), lane15)
                        res = jnp.where(lanes == j,
                                        1.0 - bsum * _INV3SQ, res)
                    tvb[pl.ds(g16 * 16, 16)] = res
                    return ecarry

                lax.fori_loop(0, K // 16, tri16, 0)
                pltpu.sync_copy(tvb, tv_ref.at[pl.ds(wid * TPW + b * K, K)])
            return carry

        lax.fori_loop(0, TB // 2, blk2, 0)

        dbase = wid * (B_ALIGN // NW)
        pltpu.sync_copy(di_ref.at[pl.ds(dbase, B_ALIGN // NW)],
                        tib.at[pl.ds(0, B_ALIGN // NW)])
        pltpu.sync_copy(g_ref.at[tib.at[pl.ds(0, B_ALIGN // NW)]], fbb)
        pltpu.sync_copy(fbb, dout_ref.at[pl.ds(dbase, B_ALIGN // NW)])

        rbase = wid * (B_REL // NW)
        pltpu.sync_copy(ri_ref.at[pl.ds(rbase, B_REL // NW)], rlx)
        pltpu.sync_copy(r_ref.at[rlx], rbb)
        pltpu.sync_copy(rbb, rout_ref.at[pl.ds(rbase, B_REL // NW)])

    return body(g, g_bf, rnorm, r_bf, ti, d_idx, rl_idx)


def _pad_idx(x, n):
    x = x.astype(jnp.int32)
    return jnp.concatenate([x, jnp.zeros((n - x.shape[0],), jnp.int32)])


def _gat_graph(x, edge_index, wcats, u1s, u2s):
    src = edge_index[0].astype(jnp.int32)
    dst = edge_index[1].astype(jnp.int32)
    loop = jnp.arange(N, dtype=jnp.int32)
    padv = jnp.full((EPAD - ETOT,), N, jnp.int32)
    src_all = jnp.concatenate([src, loop, padv]).reshape(NW * EB, KE)
    dst_all = jnp.concatenate([dst, loop, padv]).reshape(NW * EB, KE)
    pe = jnp.stack([src_all, dst_all], axis=1).reshape(NW * EB * 2 * KE)

    xp = jnp.concatenate([x, jnp.zeros((NP - N, DIM), jnp.float32)])
    wh, s1, s2 = _proj(xp, wcats[0], u1s[0], u2s[0])
    acc = _edge_pass(pe, wh, s1, s2)
    wh, s1, s2 = _combine_elu_proj(acc, wcats[1], u1s[1], u2s[1])
    acc = _edge_pass(pe, wh, s1, s2)
    return _combine_norm(acc)


def kernel(sr_data, tg_data, sr_rel_data, tg_rel_data, triples_sr_h, triples_sr_t, triples_sr_r, triples_tg_h, triples_tg_t, triples_tg_r, edge_index_sr, edge_index_tg, ent_emb_sr, ent_emb_tg, rel_emb_sr, rel_emb_tg, gat_W, gat_a_src, gat_a_dst):
    wcats, u1s, u2s = [], [], []
    pidx = np.array([(j % NHEADS) * DH + j // NHEADS for j in range(DIM)])
    for l in range(NUM_LAYER):
        w = gat_W[l]
        wcat = w.transpose(1, 2, 0).reshape(DIM, DIM)
        us = jnp.einsum('hdk,hk->dh', w, gat_a_src[l])
        ud = jnp.einsum('hdk,hk->dh', w, gat_a_dst[l])
        u1 = jnp.concatenate([us, us], axis=1)
        u2 = jnp.concatenate([ud, ud], axis=1)
        if l > 0:
            wcat, u1, u2 = wcat[pidx], u1[pidx], u2[pidx]
        wcats.append(wcat)
        u1s.append(u1)
        u2s.append(u2)

    g_sr = _gat_graph(ent_emb_sr, edge_index_sr, wcats, u1s, u2s)
    g_tg = _gat_graph(ent_emb_tg, edge_index_tg, wcats, u1s, u2s)

    rels = _relnorm(jnp.concatenate([rel_emb_sr, rel_emb_tg]))
    r_sr, r_tg = rels[:R], rels[R:]

    def pack_ti(h, t, r):
        arrs = [_pad_idx(x, TP).reshape(NW, TB, K) for x in (h, t, r)]
        return jnp.stack(arrs, axis=2).reshape(TP * 3)

    tv_sr, sr_data_repre, sr_rel_repre = _transe_gather(
        g_sr, g_sr.astype(jnp.bfloat16), r_sr, r_sr.astype(jnp.bfloat16),
        pack_ti(triples_sr_h, triples_sr_t, triples_sr_r),
        sr_data.astype(jnp.int32), sr_rel_data.astype(jnp.int32))
    tv_tg, tg_data_repre, tg_rel_repre = _transe_gather(
        g_tg, g_tg.astype(jnp.bfloat16), r_tg, r_tg.astype(jnp.bfloat16),
        pack_ti(triples_tg_h, triples_tg_t, triples_tg_r),
        tg_data.astype(jnp.int32), tg_rel_data.astype(jnp.int32))

    transe_tv = jnp.concatenate([tv_sr[:T], tv_tg[:T]])
    return (sr_data_repre, tg_data_repre, sr_rel_repre, tg_rel_repre, transe_tv)
